# baseline (speedup 1.0000x reference)
import functools
import math

import jax
import jax.numpy as jnp
from jax import lax
from jax.experimental import pallas as pl
from jax.experimental.pallas import tpu as pltpu
from jax.experimental.pallas import tpu_sc as plsc

VOCAB = 100000
EMB = 128
SCALE = math.sqrt(float(EMB))

NC = 2
NS = 16
NW = NC * NS
LANES = 16

B = 4096 * 50
BPW = B // NW
CHUNK = 128
NCHUNK = BPW // CHUNK


@functools.partial(
    pl.kernel,
    out_type=jax.ShapeDtypeStruct((B, EMB), jnp.bfloat16),
    mesh=plsc.VectorSubcoreMesh(core_axis_name="c", subcore_axis_name="s"),
    compiler_params=pltpu.CompilerParams(needs_layout_passes=False),
    scratch_types=[
        pltpu.VMEM((NCHUNK, CHUNK), jnp.int32),
        pltpu.VMEM((CHUNK, EMB), jnp.float32),
        pltpu.VMEM((CHUNK, EMB), jnp.float32),
        pltpu.VMEM((CHUNK, EMB), jnp.bfloat16),
        pltpu.VMEM((CHUNK, EMB), jnp.bfloat16),
        pltpu.SemaphoreType.DMA,
        pltpu.SemaphoreType.DMA,
    ],
)
def _sc_gather(idx_hbm, table_hbm, out_hbm, idx_v, rows0, rows1, pk0, pk1,
               sem0, sem1):
    wid = lax.axis_index("s") * NC + lax.axis_index("c")
    pltpu.sync_copy(idx_hbm.at[wid], idx_v)

    ev = jax.lax.iota(jnp.int32, 16) * 2
    od = ev + 1

    def start_gather(g, buf, sem):
        pltpu.async_copy(table_hbm.at[idx_v.at[g]], buf, sem)

    def wait_gather(g, buf, sem):
        pltpu.make_async_copy(table_hbm.at[idx_v.at[g]], buf, sem).wait()

    def pack_rows(src, dst):
        def row_body(r, carry):
            row = jnp.full((16,), r, dtype=jnp.int32)
            for c in range(EMB // 32):
                a = plsc.load_gather(src, [row, ev + c * 32]) * SCALE
                b = plsc.load_gather(src, [row, od + c * 32]) * SCALE
                dst[r, pl.ds(c * 32, 32)] = plsc.pack(
                    a, b, format=plsc.PackFormat.INTERLEAVED)
            return carry

        lax.fori_loop(0, CHUNK, row_body, 0)

    def write_out(g, buf):
        pltpu.sync_copy(buf, out_hbm.at[pl.ds(wid * BPW + g * CHUNK, CHUNK)])

    start_gather(0, rows0, sem0)

    def body(h, carry):
        start_gather(2 * h + 1, rows1, sem1)
        wait_gather(2 * h, rows0, sem0)
        pack_rows(rows0, pk0)
        write_out(2 * h, pk0)

        @pl.when(h + 1 < NCHUNK // 2)
        def _():
            start_gather(2 * h + 2, rows0, sem0)

        wait_gather(2 * h + 1, rows1, sem1)
        pack_rows(rows1, pk1)
        write_out(2 * h + 1, pk1)
        return carry

    lax.fori_loop(0, NCHUNK // 2, body, 0)


def kernel(tokens, table):
    idx = tokens.reshape(NW, NCHUNK, CHUNK).astype(jnp.int32)
    out16 = _sc_gather(idx, table)
    return out16.astype(jnp.float32).reshape(4096, 50, EMB)


# speedup vs baseline: 1.2090x; 1.2090x over previous
import functools
import math

import jax
import jax.numpy as jnp
from jax import lax
from jax.experimental import pallas as pl
from jax.experimental.pallas import tpu as pltpu
from jax.experimental.pallas import tpu_sc as plsc

VOCAB = 100000
EMB = 128
SCALE = math.sqrt(float(EMB))

NC = 2
NS = 16
NW = NC * NS
LANES = 16

NSEQ = 4096
SEQ = 50
SPW = NSEQ // NW
SEQ_PER_CHUNK = 2
CHUNK_ROWS = SEQ_PER_CHUNK * SEQ
NCHUNK = SPW // SEQ_PER_CHUNK
EMB_VREGS = EMB // LANES


@functools.partial(
    pl.kernel,
    out_type=jax.ShapeDtypeStruct((NSEQ, SEQ, EMB), jnp.float32),
    mesh=plsc.VectorSubcoreMesh(core_axis_name="c", subcore_axis_name="s"),
    scratch_types=[
        pltpu.VMEM((NCHUNK, CHUNK_ROWS), jnp.int32),
        pltpu.VMEM((CHUNK_ROWS, EMB), jnp.float32),
        pltpu.VMEM((CHUNK_ROWS, EMB), jnp.float32),
        pltpu.VMEM((SEQ_PER_CHUNK, SEQ, EMB), jnp.float32),
        pltpu.VMEM((SEQ_PER_CHUNK, SEQ, EMB), jnp.float32),
        pltpu.SemaphoreType.DMA,
        pltpu.SemaphoreType.DMA,
        pltpu.SemaphoreType.DMA,
        pltpu.SemaphoreType.DMA,
    ],
)
def _sc_gather(idx_hbm, table_hbm, out_hbm, idx_v, rows0, rows1, wb0, wb1,
               sem0, sem1, ws0, ws1):
    wid = lax.axis_index("s") * NC + lax.axis_index("c")
    pltpu.sync_copy(idx_hbm.at[wid], idx_v)

    def start_gather(g, buf, sem):
        pltpu.async_copy(table_hbm.at[idx_v.at[g]], buf, sem)

    def wait_gather(g, buf, sem):
        pltpu.make_async_copy(table_hbm.at[idx_v.at[g]], buf, sem).wait()

    def scale_rows(src, dst):
        def row_body(r, carry):
            for c in range(EMB_VREGS):
                sl = pl.ds(c * LANES, LANES)
                s = r // SEQ
                t = r % SEQ
                dst[s, t, sl] = src[r, sl] * SCALE
            return carry

        lax.fori_loop(0, CHUNK_ROWS, row_body, 0)

    def out_slot(g):
        return out_hbm.at[pl.ds(wid * SPW + g * SEQ_PER_CHUNK, SEQ_PER_CHUNK)]

    def start_write(g, buf, wsem):
        pltpu.async_copy(buf, out_slot(g), wsem)

    def wait_write(g, buf, wsem):
        pltpu.make_async_copy(buf, out_slot(g), wsem).wait()

    start_gather(0, rows0, sem0)

    def body(h, carry):
        start_gather(2 * h + 1, rows1, sem1)
        wait_gather(2 * h, rows0, sem0)

        @pl.when(h > 0)
        def _():
            wait_write(2 * h - 2, wb0, ws0)

        scale_rows(rows0, wb0)
        start_write(2 * h, wb0, ws0)

        @pl.when(h + 1 < NCHUNK // 2)
        def _():
            start_gather(2 * h + 2, rows0, sem0)

        wait_gather(2 * h + 1, rows1, sem1)

        @pl.when(h > 0)
        def _():
            wait_write(2 * h - 1, wb1, ws1)

        scale_rows(rows1, wb1)
        start_write(2 * h + 1, wb1, ws1)
        return carry

    lax.fori_loop(0, NCHUNK // 2, body, 0)
    wait_write(NCHUNK - 2, wb0, ws0)
    wait_write(NCHUNK - 1, wb1, ws1)


def kernel(tokens, table):
    idx = tokens.reshape(NW, NCHUNK, CHUNK_ROWS).astype(jnp.int32)
    return _sc_gather(idx, table)


# speedup vs baseline: 2.2550x; 1.8652x over previous
import functools
import math

import jax
import jax.numpy as jnp
from jax import lax
from jax.experimental import pallas as pl
from jax.experimental.pallas import tpu as pltpu
from jax.experimental.pallas import tpu_sc as plsc

VOCAB = 100000
EMB = 128
SCALE = math.sqrt(float(EMB))

NC = 2
NS = 16
NW = NC * NS
LANES = 16

NSEQ = 4096
SEQ = 50
SPW = NSEQ // NW
SEQ_PER_CHUNK = 2
CHUNK_ROWS = SEQ_PER_CHUNK * SEQ
NCHUNK = SPW // SEQ_PER_CHUNK
EMB_VREGS = EMB // LANES


@functools.partial(
    pl.kernel,
    out_type=jax.ShapeDtypeStruct((NSEQ, SEQ, EMB), jnp.float32),
    mesh=plsc.VectorSubcoreMesh(core_axis_name="c", subcore_axis_name="s"),
    scratch_types=[
        pltpu.VMEM((NCHUNK, CHUNK_ROWS), jnp.int32),
        pltpu.VMEM((CHUNK_ROWS, EMB), jnp.float32),
        pltpu.VMEM((CHUNK_ROWS, EMB), jnp.float32),
        pltpu.SemaphoreType.DMA,
        pltpu.SemaphoreType.DMA,
        pltpu.SemaphoreType.DMA,
        pltpu.SemaphoreType.DMA,
    ],
)
def _sc_gather(idx_hbm, table_hbm, out_hbm, idx_v, rows0, rows1,
               sem0, sem1, ws0, ws1):
    wid = lax.axis_index("s") * NC + lax.axis_index("c")
    pltpu.sync_copy(idx_hbm.at[wid], idx_v)

    def start_gather(g, buf, sem):
        pltpu.async_copy(table_hbm.at[idx_v.at[g]], buf, sem)

    def wait_gather(g, buf, sem):
        pltpu.make_async_copy(table_hbm.at[idx_v.at[g]], buf, sem).wait()

    def scale_rows(buf):
        def row_body(r, carry):
            for c in range(EMB_VREGS):
                sl = pl.ds(c * LANES, LANES)
                buf[r, sl] = buf[r, sl] * SCALE
            return carry

        lax.fori_loop(0, CHUNK_ROWS, row_body, 0)

    def start_write(g, buf, wsem):
        b0 = wid * SPW + g * SEQ_PER_CHUNK
        pltpu.async_copy(buf.at[pl.ds(0, SEQ)], out_hbm.at[b0], wsem)
        pltpu.async_copy(buf.at[pl.ds(SEQ, SEQ)], out_hbm.at[b0 + 1], wsem)

    def wait_write(g, buf, wsem):
        b0 = wid * SPW + g * SEQ_PER_CHUNK
        pltpu.make_async_copy(buf.at[pl.ds(0, SEQ)], out_hbm.at[b0], wsem).wait()
        pltpu.make_async_copy(buf.at[pl.ds(SEQ, SEQ)], out_hbm.at[b0 + 1], wsem).wait()

    start_gather(0, rows0, sem0)

    def body(h, carry):
        @pl.when(h > 0)
        def _():
            wait_write(2 * h - 1, rows1, ws1)

        start_gather(2 * h + 1, rows1, sem1)
        wait_gather(2 * h, rows0, sem0)
        scale_rows(rows0)
        start_write(2 * h, rows0, ws0)

        @pl.when(h + 1 < NCHUNK // 2)
        def _():
            wait_write(2 * h, rows0, ws0)
            start_gather(2 * h + 2, rows0, sem0)

        wait_gather(2 * h + 1, rows1, sem1)
        scale_rows(rows1)
        start_write(2 * h + 1, rows1, ws1)
        return carry

    lax.fori_loop(0, NCHUNK // 2, body, 0)
    wait_write(NCHUNK - 2, rows0, ws0)
    wait_write(NCHUNK - 1, rows1, ws1)


def kernel(tokens, table):
    idx = tokens.reshape(NW, NCHUNK, CHUNK_ROWS).astype(jnp.int32)
    return _sc_gather(idx, table)


# speedup vs baseline: 2.4493x; 1.0862x over previous
import functools
import math

import jax
import jax.numpy as jnp
from jax import lax
from jax.experimental import pallas as pl
from jax.experimental.pallas import tpu as pltpu
from jax.experimental.pallas import tpu_sc as plsc

VOCAB = 100000
EMB = 128
SCALE = math.sqrt(float(EMB))

NC = 2
NS = 16
NW = NC * NS
LANES = 16

NSEQ = 4096
SEQ = 50
SPW = NSEQ // NW
SEQ_PER_CHUNK = 2
CHUNK_ROWS = SEQ_PER_CHUNK * SEQ
NCHUNK = SPW // SEQ_PER_CHUNK
EMB_VREGS = EMB // LANES
NBUF = 8


@functools.partial(
    pl.kernel,
    out_type=jax.ShapeDtypeStruct((NSEQ, SEQ, EMB), jnp.float32),
    mesh=plsc.VectorSubcoreMesh(core_axis_name="c", subcore_axis_name="s"),
    scratch_types=[
        pltpu.VMEM((NCHUNK, CHUNK_ROWS), jnp.int32),
    ] + [pltpu.VMEM((CHUNK_ROWS, EMB), jnp.float32)] * NBUF
      + [pltpu.SemaphoreType.DMA] * (2 * NBUF),
)
def _sc_gather(idx_hbm, table_hbm, out_hbm, idx_v, *rest):
    bufs = rest[:NBUF]
    gsem = rest[NBUF:2 * NBUF]
    wsem = rest[2 * NBUF:]

    wid = lax.axis_index("s") * NC + lax.axis_index("c")
    pltpu.sync_copy(idx_hbm.at[wid], idx_v)

    def start_gather(g, buf, sem):
        pltpu.async_copy(table_hbm.at[idx_v.at[g]], buf, sem)

    def wait_gather(g, buf, sem):
        pltpu.make_async_copy(table_hbm.at[idx_v.at[g]], buf, sem).wait()

    def scale_rows(buf):
        def row_body(r, carry):
            for c in range(EMB_VREGS):
                sl = pl.ds(c * LANES, LANES)
                buf[r, sl] = buf[r, sl] * SCALE
            return carry

        lax.fori_loop(0, CHUNK_ROWS, row_body, 0)

    def start_write(g, buf, sem):
        b = wid * SPW + g * SEQ_PER_CHUNK
        pltpu.async_copy(buf.at[pl.ds(0, SEQ)], out_hbm.at[b], sem)
        pltpu.async_copy(buf.at[pl.ds(SEQ, SEQ)], out_hbm.at[b + 1], sem)

    def wait_write(g, buf, sem):
        b = wid * SPW + g * SEQ_PER_CHUNK
        pltpu.make_async_copy(buf.at[pl.ds(0, SEQ)], out_hbm.at[b], sem).wait()
        pltpu.make_async_copy(buf.at[pl.ds(SEQ, SEQ)], out_hbm.at[b + 1], sem).wait()

    for j in range(NBUF - 1):
        start_gather(j, bufs[j], gsem[j])

    def body(h, carry):
        for j in range(NBUF):
            g = NBUF * h + j
            jp = (j + NBUF - 1) % NBUF
            wait_gather(g, bufs[j], gsem[j])
            scale_rows(bufs[j])
            start_write(g, bufs[j], wsem[j])

            @pl.when(g + NBUF - 1 < NCHUNK)
            def _():
                @pl.when(g >= 1)
                def _():
                    wait_write(g - 1, bufs[jp], wsem[jp])

                start_gather(g + NBUF - 1, bufs[jp], gsem[jp])
        return carry

    lax.fori_loop(0, NCHUNK // NBUF, body, 0)
    for j in range(NBUF):
        wait_write(NCHUNK - NBUF + j, bufs[j], wsem[j])


def kernel(tokens, table):
    idx = tokens.reshape(NW, NCHUNK, CHUNK_ROWS).astype(jnp.int32)
    return _sc_gather(idx, table)
